# Initial kernel scaffold; baseline (speedup 1.0000x reference)
import jax
import jax.numpy as jnp
from jax.experimental import pallas as pl


def kernel(feature, params, edge_index, batch):
    raise NotImplementedError("write your pallas kernel here")


# speedup vs baseline: 2.2930x; 2.2930x over previous
import functools

import jax
import jax.numpy as jnp
from jax import lax
from jax.experimental import pallas as pl
from jax.experimental.pallas import tpu as pltpu

N = 10000
D = 128
E = 320000
G = 128
EPS = 1e-5


def _init_body(deg_ref, feat_ref, wemb_ref, bemb_ref, w1_ref,
               h0_ref, hwp1_ref, dinv_ref):
    dinv = lax.rsqrt(deg_ref[...])
    h0 = jnp.dot(feat_ref[...], wemb_ref[...],
                 preferred_element_type=jnp.float32) + bemb_ref[...]
    h0_ref[...] = h0
    dinv_ref[...] = dinv
    hwp1_ref[...] = jnp.dot(h0, w1_ref[...],
                            preferred_element_type=jnp.float32) * dinv


def _finish_body(hprev_ref, hwp_ref, part_ref, dinv_ref, b_ref, g_ref,
                 beta_ref, wn_ref, h_ref, hwpn_ref):
    dinv = dinv_ref[...]
    s = part_ref[...] + hwp_ref[...]
    agg = s * dinv + b_ref[...]
    mu = jnp.mean(agg, axis=0, keepdims=True)
    var = jnp.mean((agg - mu) ** 2, axis=0, keepdims=True)
    hbn = (agg - mu) * lax.rsqrt(var + EPS) * g_ref[...] + beta_ref[...]
    h = hprev_ref[...] + jnp.maximum(hbn, 0.0)
    h_ref[...] = h
    if hwpn_ref is not None:
        hwpn_ref[...] = jnp.dot(h, wn_ref[...],
                                preferred_element_type=jnp.float32) * dinv


def _final_body(hprev_ref, hwp_ref, part_ref, dinv_ref, b_ref, g_ref,
                beta_ref, batch_ref, w0_ref, b0_ref, w1_ref, b1_ref,
                w2_ref, b2_ref, out_ref):
    dinv = dinv_ref[...]
    s = part_ref[...] + hwp_ref[...]
    agg = s * dinv + b_ref[...]
    mu = jnp.mean(agg, axis=0, keepdims=True)
    var = jnp.mean((agg - mu) ** 2, axis=0, keepdims=True)
    hbn = (agg - mu) * lax.rsqrt(var + EPS) * g_ref[...] + beta_ref[...]
    h = hprev_ref[...] + jnp.maximum(hbn, 0.0)
    row_ids = lax.broadcasted_iota(jnp.int32, (G, N), 0)
    oh = (row_ids == batch_ref[...]).astype(jnp.float32)
    sums = jnp.dot(oh, h, preferred_element_type=jnp.float32)
    counts = jnp.sum(oh, axis=1, keepdims=True)
    hg = sums / jnp.maximum(counts, 1.0)
    hg = jnp.maximum(jnp.dot(hg, w0_ref[...],
                             preferred_element_type=jnp.float32) + b0_ref[...], 0.0)
    hg = jnp.maximum(jnp.dot(hg, w1_ref[...],
                             preferred_element_type=jnp.float32) + b1_ref[...], 0.0)
    out_ref[...] = jnp.dot(hg, w2_ref[...],
                           preferred_element_type=jnp.float32) + b2_ref[...]


def _tc_call(body, out_shapes, *args):
    return pl.pallas_call(
        body,
        out_shape=out_shapes,
    )(*args)


def kernel(feature, params, edge_index, batch):
    src = edge_index[0]
    dst = edge_index[1]

    deg = jax.ops.segment_sum(jnp.ones((E,), jnp.float32), dst,
                              num_segments=N) + 1.0
    deg = deg[:, None]

    wemb, bemb = params["emb"]
    gcn = params["gcn"]

    h0, hwp1, dinv = _tc_call(
        _init_body,
        [jax.ShapeDtypeStruct((N, D), jnp.float32),
         jax.ShapeDtypeStruct((N, D), jnp.float32),
         jax.ShapeDtypeStruct((N, 1), jnp.float32)],
        deg, feature, wemb, bemb[None, :], gcn[0]["W"])

    h, hwp = h0, hwp1
    for l in range(3):
        part = jax.ops.segment_sum(jnp.take(hwp, src, axis=0), dst,
                                   num_segments=N)
        lyr = gcn[l]
        wn = gcn[l + 1]["W"]
        h, hwp = pl.pallas_call(
            _finish_body,
            out_shape=[jax.ShapeDtypeStruct((N, D), jnp.float32),
                       jax.ShapeDtypeStruct((N, D), jnp.float32)],
        )(h, hwp, part, dinv, lyr["b"][None, :], lyr["gamma"][None, :],
          lyr["beta"][None, :], wn)

    part = jax.ops.segment_sum(jnp.take(hwp, src, axis=0), dst,
                               num_segments=N)
    lyr = gcn[3]
    (w0, b0), (w1, b1), (w2, b2) = params["mlp"]
    out = pl.pallas_call(
        _final_body,
        out_shape=jax.ShapeDtypeStruct((G, params["mlp"][2][0].shape[1]),
                                       jnp.float32),
    )(h, hwp, part, dinv, lyr["b"][None, :], lyr["gamma"][None, :],
      lyr["beta"][None, :], batch[None, :], w0, b0[None, :], w1, b1[None, :],
      w2, b2[None, :])
    return out


# speedup vs baseline: 10.0616x; 4.3879x over previous
import jax
import jax.numpy as jnp
from jax import lax
from jax.experimental import pallas as pl
from jax.experimental.pallas import tpu as pltpu
from jax.experimental.pallas import tpu_sc as plsc

N = 10000
D = 128
E = 320000
G = 128
EPS = 1e-5

NC = 2
NS = 16
NW = NC * NS
BLK = 128
NB = 79
E_PAD = NW * NB * BLK
N_PAD = 10112
RPT = N_PAD // NS

_MESH = plsc.VectorSubcoreMesh(core_axis_name="c", subcore_axis_name="s")


def _sc_edge_body(hwp_hbm, src_hbm, dst_hbm, zrow_hbm, out_hbm,
                  src_v, dst_v, rows_v, agg, sem):
    c = lax.axis_index("c")
    s = lax.axis_index("s")
    w = c * NS + s
    base = s * RPT

    pltpu.sync_copy(zrow_hbm, agg.at[pl.ds(base, RPT)])
    pltpu.sync_copy(src_hbm.at[w], src_v)
    pltpu.sync_copy(dst_hbm.at[w], dst_v)
    plsc.subcore_barrier()

    @pl.loop(0, NB)
    def _(j):
        pltpu.async_copy(hwp_hbm.at[src_v.at[j]], rows_v, sem).wait()
        pltpu.sync_copy(rows_v, agg.at[dst_v.at[j]], add=True)

    plsc.subcore_barrier()
    pltpu.sync_copy(agg.at[pl.ds(base, RPT)],
                    out_hbm.at[c, pl.ds(base, RPT)])


def _sc_edge(hwp, src3, dst3, zrow):
    f = pl.kernel(
        _sc_edge_body,
        out_type=jax.ShapeDtypeStruct((NC, N_PAD, D), jnp.float32),
        mesh=_MESH,
        scratch_types=[
            pltpu.VMEM((NB, BLK), jnp.int32),
            pltpu.VMEM((NB, BLK), jnp.int32),
            pltpu.VMEM((BLK, D), jnp.float32),
            pltpu.VMEM_SHARED((N_PAD, D), jnp.float32),
            pltpu.SemaphoreType.DMA,
        ],
    )
    return f(hwp, src3, dst3, zrow)


def _sc_deg_body(dst_hbm, zrow_hbm, out_hbm, dst_v, ones_v, agg, sem):
    c = lax.axis_index("c")
    s = lax.axis_index("s")
    w = c * NS + s
    base = s * RPT

    pltpu.sync_copy(zrow_hbm, agg.at[pl.ds(base, RPT)])
    pltpu.sync_copy(dst_hbm.at[w], dst_v)

    @pl.loop(0, BLK)
    def _(i):
        ones_v[i, :] = jnp.full((16,), 1.0, jnp.float32)

    plsc.subcore_barrier()

    @pl.loop(0, NB)
    def _(j):
        pltpu.sync_copy(ones_v, agg.at[dst_v.at[j]], add=True)

    plsc.subcore_barrier()
    pltpu.sync_copy(agg.at[pl.ds(base, RPT)],
                    out_hbm.at[c, pl.ds(base, RPT)])


def _sc_deg(dst3, zrow16):
    f = pl.kernel(
        _sc_deg_body,
        out_type=jax.ShapeDtypeStruct((NC, N_PAD, 16), jnp.float32),
        mesh=_MESH,
        scratch_types=[
            pltpu.VMEM((NB, BLK), jnp.int32),
            pltpu.VMEM((BLK, 16), jnp.float32),
            pltpu.VMEM_SHARED((N_PAD, 16), jnp.float32),
            pltpu.SemaphoreType.DMA,
        ],
    )
    return f(dst3, zrow16)


def _init_body(degp_ref, feat_ref, wemb_ref, bemb_ref, w1_ref,
               h0_ref, hwp1_ref, dinv_ref):
    dsum = jnp.sum(degp_ref[0, :N, :] + degp_ref[1, :N, :],
                   axis=1, keepdims=True)
    deg = dsum * (1.0 / 16.0) + 1.0
    dinv = lax.rsqrt(deg)
    h0 = jnp.dot(feat_ref[...], wemb_ref[...],
                 preferred_element_type=jnp.float32) + bemb_ref[...]
    h0_ref[...] = h0
    dinv_ref[...] = dinv
    hwp1_ref[...] = jnp.dot(h0, w1_ref[...],
                            preferred_element_type=jnp.float32) * dinv


def _finish_body(hprev_ref, hwp_ref, part_ref, dinv_ref, b_ref, g_ref,
                 beta_ref, wn_ref, h_ref, hwpn_ref):
    dinv = dinv_ref[...]
    s = part_ref[0, :N, :] + part_ref[1, :N, :] + hwp_ref[...]
    agg = s * dinv + b_ref[...]
    mu = jnp.mean(agg, axis=0, keepdims=True)
    var = jnp.mean((agg - mu) ** 2, axis=0, keepdims=True)
    hbn = (agg - mu) * lax.rsqrt(var + EPS) * g_ref[...] + beta_ref[...]
    h = hprev_ref[...] + jnp.maximum(hbn, 0.0)
    h_ref[...] = h
    hwpn_ref[...] = jnp.dot(h, wn_ref[...],
                            preferred_element_type=jnp.float32) * dinv


def _final_body(hprev_ref, hwp_ref, part_ref, dinv_ref, b_ref, g_ref,
                beta_ref, batch_ref, w0_ref, b0_ref, w1_ref, b1_ref,
                w2_ref, b2_ref, out_ref):
    dinv = dinv_ref[...]
    s = part_ref[0, :N, :] + part_ref[1, :N, :] + hwp_ref[...]
    agg = s * dinv + b_ref[...]
    mu = jnp.mean(agg, axis=0, keepdims=True)
    var = jnp.mean((agg - mu) ** 2, axis=0, keepdims=True)
    hbn = (agg - mu) * lax.rsqrt(var + EPS) * g_ref[...] + beta_ref[...]
    h = hprev_ref[...] + jnp.maximum(hbn, 0.0)
    row_ids = lax.broadcasted_iota(jnp.int32, (G, N), 0)
    oh = (row_ids == batch_ref[...]).astype(jnp.float32)
    sums = jnp.dot(oh, h, preferred_element_type=jnp.float32)
    counts = jnp.sum(oh, axis=1, keepdims=True)
    hg = sums / jnp.maximum(counts, 1.0)
    hg = jnp.maximum(jnp.dot(hg, w0_ref[...],
                             preferred_element_type=jnp.float32) + b0_ref[...], 0.0)
    hg = jnp.maximum(jnp.dot(hg, w1_ref[...],
                             preferred_element_type=jnp.float32) + b1_ref[...], 0.0)
    out_ref[...] = jnp.dot(hg, w2_ref[...],
                           preferred_element_type=jnp.float32) + b2_ref[...]


def kernel(feature, params, edge_index, batch):
    src = edge_index[0]
    dst = edge_index[1]
    pad = E_PAD - E
    src3 = jnp.concatenate(
        [src, jnp.zeros((pad,), jnp.int32)]).reshape(NW, NB, BLK)
    dst3 = jnp.concatenate(
        [dst, jnp.full((pad,), N, jnp.int32)]).reshape(NW, NB, BLK)
    zrow = jnp.zeros((RPT, D), jnp.float32)
    zrow16 = jnp.zeros((RPT, 16), jnp.float32)

    degp = _sc_deg(dst3, zrow16)

    wemb, bemb = params["emb"]
    gcn = params["gcn"]

    h0, hwp1, dinv = pl.pallas_call(
        _init_body,
        out_shape=[jax.ShapeDtypeStruct((N, D), jnp.float32),
                   jax.ShapeDtypeStruct((N, D), jnp.float32),
                   jax.ShapeDtypeStruct((N, 1), jnp.float32)],
    )(degp, feature, wemb, bemb[None, :], gcn[0]["W"])

    h, hwp = h0, hwp1
    for l in range(3):
        part = _sc_edge(hwp, src3, dst3, zrow)
        lyr = gcn[l]
        wn = gcn[l + 1]["W"]
        h, hwp = pl.pallas_call(
            _finish_body,
            out_shape=[jax.ShapeDtypeStruct((N, D), jnp.float32),
                       jax.ShapeDtypeStruct((N, D), jnp.float32)],
        )(h, hwp, part, dinv, lyr["b"][None, :], lyr["gamma"][None, :],
          lyr["beta"][None, :], wn)

    part = _sc_edge(hwp, src3, dst3, zrow)
    lyr = gcn[3]
    (w0, b0), (w1, b1), (w2, b2) = params["mlp"]
    out = pl.pallas_call(
        _final_body,
        out_shape=jax.ShapeDtypeStruct((G, params["mlp"][2][0].shape[1]),
                                       jnp.float32),
    )(h, hwp, part, dinv, lyr["b"][None, :], lyr["gamma"][None, :],
      lyr["beta"][None, :], batch[None, :], w0, b0[None, :], w1, b1[None, :],
      w2, b2[None, :])
    return out
